# Initial kernel scaffold; baseline (speedup 1.0000x reference)
import jax
import jax.numpy as jnp
from jax.experimental import pallas as pl


def kernel(edge_index, edge_w, gumbel_noise, aspect_emb, center_emb, attn_W, attn_b, W1, b1, W2, b2, W3, b3, bn_gamma, bn_beta, lin_W, lin_b):
    raise NotImplementedError("write your pallas kernel here")


# speedup vs baseline: 4.4846x; 4.4846x over previous
import functools

import jax
import jax.numpy as jnp
from jax import lax
from jax.experimental import pallas as pl
from jax.experimental.pallas import tpu as pltpu
from jax.experimental.pallas import tpu_sc as plsc

f32 = jnp.float32
i32 = jnp.int32

NC = 2
NS = 16
NW = NC * NS
L = 16


def _perm(x, idx):
  dn = lax.GatherDimensionNumbers(
      offset_dims=(), collapsed_slice_dims=(0,), start_index_map=(0,))
  return lax.gather(x, idx[:, None], dn, slice_sizes=(1,),
                    mode=lax.GatherScatterMode.PROMISE_IN_BOUNDS)


def _mesh():
  return plsc.VectorSubcoreMesh(core_axis_name="c", subcore_axis_name="s")


def _degrees(src, dst, n_nodes):
  E = src.shape[0]
  ec = E // NW
  C = 80
  nch = ec // C
  NP = 2 * ((n_nodes + 639) // 640) * 640
  stripe = NP // NS

  def body(src_hbm, dst_hbm, out_hbm, sidx, didx, ones_v, zb, deg_sh):
    cid = lax.axis_index("c")
    sid = lax.axis_index("s")
    wid = sid * NC + cid

    def fill(i, _):
      ones_v[pl.ds(i * L, L)] = jnp.full((L,), 1.0, f32)
      return 0
    lax.fori_loop(0, C // L, fill, 0)

    def zfill(i, _):
      zb[pl.ds(i * L, L)] = jnp.zeros((L,), f32)
      return 0
    lax.fori_loop(0, stripe // L, zfill, 0)
    pltpu.sync_copy(zb, deg_sh.at[pl.ds(sid * stripe, stripe)])
    plsc.subcore_barrier()

    def chunk(ch, _):
      eb = wid * ec + ch * C
      pltpu.sync_copy(src_hbm.at[pl.ds(eb, C)], sidx)
      pltpu.sync_copy(dst_hbm.at[pl.ds(eb, C)], didx)

      def shift(q, _):
        didx[pl.ds(q * L, L)] = didx[pl.ds(q * L, L)] + n_nodes
        return 0
      lax.fori_loop(0, C // L, shift, 0)
      pltpu.sync_copy(ones_v, deg_sh.at[sidx], add=True)
      pltpu.sync_copy(ones_v, deg_sh.at[didx], add=True)
      return 0
    lax.fori_loop(0, nch, chunk, 0)
    plsc.subcore_barrier()
    o0 = pl.multiple_of(cid * NP + sid * stripe, 128)
    pltpu.sync_copy(deg_sh.at[pl.ds(pl.multiple_of(sid * stripe, 128), stripe)],
                    out_hbm.at[pl.ds(o0, stripe)])

  return pl.kernel(
      body,
      out_type=jax.ShapeDtypeStruct((NC * NP,), f32),
      mesh=_mesh(),
      scratch_types=[
          pltpu.VMEM((C,), i32),
          pltpu.VMEM((C,), i32),
          pltpu.VMEM((C,), f32),
          pltpu.VMEM((stripe,), f32),
          pltpu.VMEM_SHARED((NP,), f32),
      ],
  )(src, dst)


def _edge_weight(w0, w1, gnflat, aspTX):
  E = w0.shape[0]
  DM = aspTX.shape[1]
  D = 128
  ec = E // NW
  C = 80
  nch = ec // C

  def body(w0_hbm, w1_hbm, gn_hbm, asp_hbm, ew_hbm,
           w0v, w1v, gnv, ab0, ab1, ewb, sem):
    cid = lax.axis_index("c")
    sid = lax.axis_index("s")
    wid = sid * NC + cid
    iot = lax.iota(i32, L)
    kv = iot & 3

    def chunk(ch, _):
      eb = wid * ec + ch * C
      pltpu.sync_copy(w0_hbm.at[pl.ds(eb, C)], w0v)
      pltpu.sync_copy(w1_hbm.at[pl.ds(eb, C)], w1v)
      pltpu.sync_copy(gn_hbm.at[pl.ds(eb * 4, C * 4)], gnv)
      cp2 = pltpu.async_copy(asp_hbm.at[w0v], ab0, sem)
      cp3 = pltpu.async_copy(asp_hbm.at[w1v], ab1, sem)
      cp2.wait(); cp3.wait()

      def edge(je, _):
        s0 = ab0[je, pl.ds(4 * D, L)]
        s1 = ab1[je, pl.ds(4 * D, L)]
        gg = gnv[pl.ds((je >> 2) * L, L)]
        gn_e = _perm(gg, kv + 4 * (je & 3))
        t = (s0 + s1 + gn_e) * 2.0
        m = jnp.maximum(t, _perm(t, iot ^ 1))
        m = jnp.maximum(m, _perm(m, iot ^ 2))
        p = jnp.exp(t - m)
        q = p + _perm(p, iot ^ 1)
        q = q + _perm(q, iot ^ 2)
        attn = p / q
        for d in range(D // L):
          acc = None
          for k in range(4):
            off = k * D + d * L
            term = ab0[je, pl.ds(off, L)] + ab1[je, pl.ds(off, L)]
            a = attn[k]
            acc = term * a if acc is None else acc + term * a
          ewb[je, pl.ds(d * L, L)] = acc
        return 0
      lax.fori_loop(0, C, edge, 0)
      pltpu.sync_copy(ewb, ew_hbm.at[pl.ds(pl.multiple_of(eb, 8), C), :])
      return 0
    lax.fori_loop(0, nch, chunk, 0)

  return pl.kernel(
      body,
      out_type=jax.ShapeDtypeStruct((E, D), f32),
      mesh=_mesh(),
      scratch_types=[
          pltpu.VMEM((C,), i32),
          pltpu.VMEM((C,), i32),
          pltpu.VMEM((4 * C,), f32),
          pltpu.VMEM((C, DM), f32),
          pltpu.VMEM((C, DM), f32),
          pltpu.VMEM((C, D), f32),
          pltpu.SemaphoreType.DMA,
      ],
  )(w0, w1, gnflat, aspTX)


def _gconv_scatter(feat, ew, src, dst):
  n_nodes, D = feat.shape
  E = src.shape[0]
  ec = E // NW
  C = 80
  nch = ec // C
  NSTRIPE = 10
  rows_pt = n_nodes // NSTRIPE
  zrows = 200

  def body(feat_hbm, ew_hbm, src_hbm, dst_hbm, out_hbm,
           sidx, didx, fv, ev, zb, agg_sh, sem):
    cid = lax.axis_index("c")
    sid = lax.axis_index("s")
    wid = sid * NC + cid

    def zfill(j, _):
      for d in range(D // L):
        zb[j, pl.ds(d * L, L)] = jnp.zeros((L,), f32)
      return 0
    lax.fori_loop(0, zrows, zfill, 0)

    @pl.when(sid < NSTRIPE)
    def _():
      for b in range(rows_pt // zrows):
        r0 = pl.multiple_of(sid * rows_pt + b * zrows, 8)
        pltpu.sync_copy(zb, agg_sh.at[pl.ds(r0, zrows), :])
    plsc.subcore_barrier()

    def chunk(ch, _):
      eb = wid * ec + ch * C
      pltpu.sync_copy(src_hbm.at[pl.ds(eb, C)], sidx)
      pltpu.sync_copy(dst_hbm.at[pl.ds(eb, C)], didx)
      pltpu.async_copy(feat_hbm.at[sidx], fv, sem).wait()
      pltpu.sync_copy(ew_hbm.at[pl.ds(pl.multiple_of(eb, 8), C), :], ev)

      def edge(je, _):
        for d in range(D // L):
          fv[je, pl.ds(d * L, L)] = fv[je, pl.ds(d * L, L)] * ev[je, pl.ds(d * L, L)]
        return 0
      lax.fori_loop(0, C, edge, 0)
      pltpu.sync_copy(fv, agg_sh.at[didx], add=True)
      return 0
    lax.fori_loop(0, nch, chunk, 0)
    plsc.subcore_barrier()

    @pl.when(sid < NSTRIPE)
    def _():
      for b in range(rows_pt // zrows):
        r0 = pl.multiple_of(sid * rows_pt + b * zrows, 8)
        pltpu.sync_copy(agg_sh.at[pl.ds(r0, zrows), :],
                        out_hbm.at[cid, pl.ds(r0, zrows), :])

  return pl.kernel(
      body,
      out_type=jax.ShapeDtypeStruct((NC, n_nodes, D), f32),
      mesh=_mesh(),
      scratch_types=[
          pltpu.VMEM((C,), i32),
          pltpu.VMEM((C,), i32),
          pltpu.VMEM((C, D), f32),
          pltpu.VMEM((C, D), f32),
          pltpu.VMEM((zrows, D), f32),
          pltpu.VMEM_SHARED((n_nodes, D), f32),
          pltpu.SemaphoreType.DMA,
      ],
  )(feat, ew, src, dst)


def _tc_pack(a3, attn_w2):
  A_, N, D = a3.shape
  blk = 1000

  def body(a_ref, w_ref, o_ref):
    a = a_ref[...]
    o_ref[:, 0:A_, :] = jnp.swapaxes(a, 0, 1)
    s = jnp.sum(a * w_ref[0], axis=-1)
    st = s.T
    row = jnp.concatenate(
        [st, st, st, st, jnp.zeros((blk, D - 4 * A_), f32)], axis=1)
    o_ref[:, A_:A_ + 1, :] = row[:, None, :]

  return pl.pallas_call(
      body,
      grid=(N // blk,),
      in_specs=[
          pl.BlockSpec((A_, blk, D), lambda i: (0, i, 0)),
          pl.BlockSpec((1, D), lambda i: (0, 0)),
      ],
      out_specs=pl.BlockSpec((blk, A_ + 1, D), lambda i: (i, 0, 0)),
      out_shape=jax.ShapeDtypeStruct((N, A_ + 1, D), f32),
  )(a3, attn_w2)


def _tc_feat1(center, odp):
  N, D = center.shape

  def body(c_ref, od_ref, o_ref):
    od = jnp.maximum(od_ref[:, 0:1] + od_ref[:, 1:2], 1.0)
    o_ref[...] = c_ref[...] * lax.rsqrt(od)

  return pl.pallas_call(
      body,
      out_shape=jax.ShapeDtypeStruct((N, D), f32),
  )(center, odp)


def _tc_dense(aggp, odp, idp, W, b2, g2, be2):
  _, N, D = aggp.shape

  def body(a_ref, od_ref, id_ref, w_ref, b_ref, g_ref, be_ref, o_ref):
    idn = lax.rsqrt(jnp.maximum(id_ref[:, 0:1] + id_ref[:, 1:2], 1.0))
    agg = (a_ref[0] + a_ref[1]) * idn
    y = jnp.dot(agg, w_ref[...], preferred_element_type=f32) + b_ref[...]
    mu = jnp.mean(y, axis=0, keepdims=True)
    var = jnp.mean((y - mu) ** 2, axis=0, keepdims=True)
    yn = (y - mu) * lax.rsqrt(var + 1e-5) * g_ref[...] + be_ref[...]
    yn = jnp.where(yn > 0, yn, jnp.exp(jnp.minimum(yn, 0.0)) - 1.0)
    odn = lax.rsqrt(jnp.maximum(od_ref[:, 0:1] + od_ref[:, 1:2], 1.0))
    o_ref[...] = yn * odn

  return pl.pallas_call(
      body,
      out_shape=jax.ShapeDtypeStruct((N, D), f32),
  )(aggp, odp, idp, W, b2, g2, be2)


def _tc_final(aggp, idp, W, b2, g2, be2, linW, linb2):
  _, N, D = aggp.shape
  NL = linW.shape[1]

  def body(a_ref, id_ref, w_ref, b_ref, g_ref, be_ref, lw_ref, lb_ref,
           h_ref, o2_ref):
    idn = lax.rsqrt(jnp.maximum(id_ref[:, 0:1] + id_ref[:, 1:2], 1.0))
    agg = (a_ref[0] + a_ref[1]) * idn
    y = jnp.dot(agg, w_ref[...], preferred_element_type=f32) + b_ref[...]
    mu = jnp.mean(y, axis=0, keepdims=True)
    var = jnp.mean((y - mu) ** 2, axis=0, keepdims=True)
    h = (y - mu) * lax.rsqrt(var + 1e-5) * g_ref[...] + be_ref[...]
    h_ref[...] = h
    o2_ref[...] = jnp.dot(h, lw_ref[...], preferred_element_type=f32) + lb_ref[...]

  return pl.pallas_call(
      body,
      out_shape=(
          jax.ShapeDtypeStruct((N, D), f32),
          jax.ShapeDtypeStruct((N, NL), f32),
      ),
  )(aggp, idp, W, b2, g2, be2, linW, linb2)


def kernel(edge_index, edge_w, gumbel_noise, aspect_emb, center_emb,
           attn_W, attn_b, W1, b1, W2, b2, W3, b3,
           bn_gamma, bn_beta, lin_W, lin_b):
  N, D = center_emb.shape
  A_ = gumbel_noise.shape[1]
  E = edge_index.shape[1]

  src = edge_index[0]
  dst = edge_index[1]
  w0 = edge_w[:, 0]
  w1 = edge_w[:, 1]
  a3 = aspect_emb.reshape(A_, N, D)
  gnflat = gumbel_noise.reshape(-1)

  degflat = _degrees(src, dst, N)
  NP = degflat.shape[0] // NC
  degout = degflat.reshape(NC, NP)
  odp = degout[:, :N].T
  idp = degout[:, N:2 * N].T

  aspTX = _tc_pack(a3, attn_W.reshape(1, D)).reshape(N, (A_ + 1) * D)
  ew = _edge_weight(w0, w1, gnflat, aspTX)

  feat = _tc_feat1(center_emb, odp)
  for (W, b) in ((W1, b1), (W2, b2)):
    aggp = _gconv_scatter(feat, ew, src, dst)
    feat = _tc_dense(aggp, odp, idp, W, b.reshape(1, D),
                     bn_gamma.reshape(1, D), bn_beta.reshape(1, D))
  aggp = _gconv_scatter(feat, ew, src, dst)
  h, out2 = _tc_final(aggp, idp, W3, b3.reshape(1, D),
                      bn_gamma.reshape(1, D), bn_beta.reshape(1, D),
                      lin_W, lin_b.reshape(1, lin_W.shape[1]))
  return (h, out2)


# speedup vs baseline: 6.7870x; 1.5134x over previous
import functools

import jax
import jax.numpy as jnp
from jax import lax
from jax.experimental import pallas as pl
from jax.experimental.pallas import tpu as pltpu
from jax.experimental.pallas import tpu_sc as plsc

f32 = jnp.float32
i32 = jnp.int32

NC = 2
NS = 16
NW = NC * NS
L = 16


def _perm(x, idx):
  dn = lax.GatherDimensionNumbers(
      offset_dims=(), collapsed_slice_dims=(0,), start_index_map=(0,))
  return lax.gather(x, idx[:, None], dn, slice_sizes=(1,),
                    mode=lax.GatherScatterMode.PROMISE_IN_BOUNDS)


def _mesh():
  return plsc.VectorSubcoreMesh(core_axis_name="c", subcore_axis_name="s")


def _degrees(src, dst, n_nodes):
  E = src.shape[0]
  ec = E // NW
  C = 80
  nch = ec // C
  NP = 2 * ((n_nodes + 639) // 640) * 640
  stripe = NP // NS

  def body(src_hbm, dst_hbm, out_hbm, sidx, didx, ones_v, zb, deg_sh):
    cid = lax.axis_index("c")
    sid = lax.axis_index("s")
    wid = sid * NC + cid

    def fill(i, _):
      ones_v[pl.ds(i * L, L)] = jnp.full((L,), 1.0, f32)
      return 0
    lax.fori_loop(0, C // L, fill, 0)

    def zfill(i, _):
      zb[pl.ds(i * L, L)] = jnp.zeros((L,), f32)
      return 0
    lax.fori_loop(0, stripe // L, zfill, 0)
    pltpu.sync_copy(zb, deg_sh.at[pl.ds(sid * stripe, stripe)])
    plsc.subcore_barrier()

    def chunk(ch, _):
      eb = wid * ec + ch * C
      pltpu.sync_copy(src_hbm.at[pl.ds(eb, C)], sidx)
      pltpu.sync_copy(dst_hbm.at[pl.ds(eb, C)], didx)

      def shift(q, _):
        didx[pl.ds(q * L, L)] = didx[pl.ds(q * L, L)] + n_nodes
        return 0
      lax.fori_loop(0, C // L, shift, 0)
      pltpu.sync_copy(ones_v, deg_sh.at[sidx], add=True)
      pltpu.sync_copy(ones_v, deg_sh.at[didx], add=True)
      return 0
    lax.fori_loop(0, nch, chunk, 0)
    plsc.subcore_barrier()
    o0 = pl.multiple_of(cid * NP + sid * stripe, 128)
    pltpu.sync_copy(deg_sh.at[pl.ds(pl.multiple_of(sid * stripe, 128), stripe)],
                    out_hbm.at[pl.ds(o0, stripe)])

  return pl.kernel(
      body,
      out_type=jax.ShapeDtypeStruct((NC * NP,), f32),
      mesh=_mesh(),
      scratch_types=[
          pltpu.VMEM((C,), i32),
          pltpu.VMEM((C,), i32),
          pltpu.VMEM((C,), f32),
          pltpu.VMEM((stripe,), f32),
          pltpu.VMEM_SHARED((NP,), f32),
      ],
  )(src, dst)


def _edge_weight(w0, w1, gnflat, aspTX):
  E = w0.shape[0]
  DM = aspTX.shape[1]
  D = 128
  ec = E // NW
  C = 40
  nch = ec // C
  npair = nch // 2

  def body(w0_hbm, w1_hbm, gn_hbm, asp_hbm, ew_hbm,
           w0va, w0vb, w1va, w1vb, gnva, gnvb,
           ab0a, ab0b, ab1a, ab1b, ewba, ewbb,
           semi0, semi1, semg0, semg1, semo0, semo1):
    w0v = (w0va, w0vb)
    w1v = (w1va, w1vb)
    gnv = (gnva, gnvb)
    ab0 = (ab0a, ab0b)
    ab1 = (ab1a, ab1b)
    ewb = (ewba, ewbb)
    cid = lax.axis_index("c")
    sid = lax.axis_index("s")
    wid = sid * NC + cid
    iot = lax.iota(i32, L)
    kv = iot & 3
    semi = (semi0, semi1)
    semg = (semg0, semg1)
    semo = (semo0, semo1)

    def ebase(g):
      return wid * ec + g * C

    def fire_idx(g, b):
      eb = ebase(g)
      pltpu.async_copy(w0_hbm.at[pl.ds(eb, C)], w0v[b], semi[b])
      pltpu.async_copy(w1_hbm.at[pl.ds(eb, C)], w1v[b], semi[b])
      pltpu.async_copy(gn_hbm.at[pl.ds(eb * 4, C * 4)], gnv[b], semi[b])

    def wait_idx(g, b):
      eb = ebase(g)
      pltpu.make_async_copy(w0_hbm.at[pl.ds(eb, C)], w0v[b], semi[b]).wait()
      pltpu.make_async_copy(w1_hbm.at[pl.ds(eb, C)], w1v[b], semi[b]).wait()
      pltpu.make_async_copy(gn_hbm.at[pl.ds(eb * 4, C * 4)], gnv[b], semi[b]).wait()

    def fire_gather(b):
      pltpu.async_copy(asp_hbm.at[w0v[b]], ab0[b], semg[b])
      pltpu.async_copy(asp_hbm.at[w1v[b]], ab1[b], semg[b])

    def wait_gather(b):
      pltpu.make_async_copy(asp_hbm.at[w0v[b]], ab0[b], semg[b]).wait()
      pltpu.make_async_copy(asp_hbm.at[w1v[b]], ab1[b], semg[b]).wait()

    def fire_store(g, b):
      eb = pl.multiple_of(ebase(g), 8)
      pltpu.async_copy(ewb[b], ew_hbm.at[pl.ds(eb, C), :], semo[b])

    def wait_store(g, b):
      eb = pl.multiple_of(ebase(g), 8)
      pltpu.make_async_copy(ewb[b], ew_hbm.at[pl.ds(eb, C), :], semo[b]).wait()

    def compute(b):
      def edge(je, _):
        s0 = ab0[b][je, pl.ds(4 * D, L)]
        s1 = ab1[b][je, pl.ds(4 * D, L)]
        gg = gnv[b][pl.ds((je >> 2) * L, L)]
        gn_e = _perm(gg, kv + 4 * (je & 3))
        t = (s0 + s1 + gn_e) * 2.0
        m = jnp.maximum(t, _perm(t, iot ^ 1))
        m = jnp.maximum(m, _perm(m, iot ^ 2))
        p = jnp.exp(t - m)
        q = p + _perm(p, iot ^ 1)
        q = q + _perm(q, iot ^ 2)
        attn = p / q
        for d in range(D // L):
          acc = None
          for k in range(4):
            off = k * D + d * L
            term = ab0[b][je, pl.ds(off, L)] + ab1[b][je, pl.ds(off, L)]
            a = attn[k]
            acc = term * a if acc is None else acc + term * a
          ewb[b][je, pl.ds(d * L, L)] = acc
        return 0
      lax.fori_loop(0, C, edge, 0)

    def step(cur, b):
      nxt = cur + 1

      @pl.when(nxt < nch)
      def _():
        wait_idx(nxt, b ^ 1)
        fire_gather(b ^ 1)
      wait_gather(b)

      @pl.when(cur >= 2)
      def _():
        wait_store(cur, b)
      compute(b)
      fire_store(cur, b)

      @pl.when(cur + 2 < nch)
      def _():
        fire_idx(cur + 2, b)

    fire_idx(0, 0)
    fire_idx(1, 1)
    wait_idx(0, 0)
    fire_gather(0)

    def pair(i, _):
      step(2 * i, 0)
      step(2 * i + 1, 1)
      return 0
    lax.fori_loop(0, npair, pair, 0)
    wait_store(nch - 2, 0)
    wait_store(nch - 1, 1)

  return pl.kernel(
      body,
      out_type=jax.ShapeDtypeStruct((E, D), f32),
      mesh=_mesh(),
      scratch_types=[
          pltpu.VMEM((C,), i32),
          pltpu.VMEM((C,), i32),
          pltpu.VMEM((C,), i32),
          pltpu.VMEM((C,), i32),
          pltpu.VMEM((4 * C,), f32),
          pltpu.VMEM((4 * C,), f32),
          pltpu.VMEM((C, DM), f32),
          pltpu.VMEM((C, DM), f32),
          pltpu.VMEM((C, DM), f32),
          pltpu.VMEM((C, DM), f32),
          pltpu.VMEM((C, D), f32),
          pltpu.VMEM((C, D), f32),
          pltpu.SemaphoreType.DMA,
          pltpu.SemaphoreType.DMA,
          pltpu.SemaphoreType.DMA,
          pltpu.SemaphoreType.DMA,
          pltpu.SemaphoreType.DMA,
          pltpu.SemaphoreType.DMA,
      ],
  )(w0, w1, gnflat, aspTX)


def _gconv_scatter(feat, ew, src, dst):
  n_nodes, D = feat.shape
  E = src.shape[0]
  ec = E // NW
  C = 40
  nch = ec // C
  NSTRIPE = 10
  rows_pt = n_nodes // NSTRIPE
  zrows = 40

  def body(feat_hbm, ew_hbm, src_hbm, dst_hbm, out_hbm,
           sidxa, sidxb, didxa, didxb, fva, fvb, eva, evb, zb, agg_sh,
           semi0, semi1, semg0, semg1, seme0, seme1):
    sidx = (sidxa, sidxb)
    didx = (didxa, didxb)
    fv = (fva, fvb)
    ev = (eva, evb)
    cid = lax.axis_index("c")
    sid = lax.axis_index("s")
    wid = sid * NC + cid

    def zfill(j, _):
      for d in range(D // L):
        zb[j, pl.ds(d * L, L)] = jnp.zeros((L,), f32)
      return 0
    lax.fori_loop(0, zrows, zfill, 0)

    @pl.when(sid < NSTRIPE)
    def _():
      for b in range(rows_pt // zrows):
        r0 = pl.multiple_of(sid * rows_pt + b * zrows, 8)
        pltpu.sync_copy(zb, agg_sh.at[pl.ds(r0, zrows), :])
    plsc.subcore_barrier()

    semi = (semi0, semi1)
    semg = (semg0, semg1)
    seme = (seme0, seme1)

    def ebase(g):
      return wid * ec + g * C

    def fire_idx(g, b):
      eb = ebase(g)
      pltpu.async_copy(src_hbm.at[pl.ds(eb, C)], sidx[b], semi[b])
      pltpu.async_copy(dst_hbm.at[pl.ds(eb, C)], didx[b], semi[b])

    def wait_idx(g, b):
      eb = ebase(g)
      pltpu.make_async_copy(src_hbm.at[pl.ds(eb, C)], sidx[b], semi[b]).wait()
      pltpu.make_async_copy(dst_hbm.at[pl.ds(eb, C)], didx[b], semi[b]).wait()

    def fire_gather(g, b):
      eb = pl.multiple_of(ebase(g), 8)
      pltpu.async_copy(feat_hbm.at[sidx[b]], fv[b], semg[b])
      pltpu.async_copy(ew_hbm.at[pl.ds(eb, C), :], ev[b], seme[b])

    def wait_gather(g, b):
      eb = pl.multiple_of(ebase(g), 8)
      pltpu.make_async_copy(feat_hbm.at[sidx[b]], fv[b], semg[b]).wait()
      pltpu.make_async_copy(ew_hbm.at[pl.ds(eb, C), :], ev[b], seme[b]).wait()

    def compute_scatter(b):
      def edge(je, _):
        for d in range(D // L):
          fv[b][je, pl.ds(d * L, L)] = (
              fv[b][je, pl.ds(d * L, L)] * ev[b][je, pl.ds(d * L, L)])
        return 0
      lax.fori_loop(0, C, edge, 0)
      pltpu.sync_copy(fv[b], agg_sh.at[didx[b]], add=True)

    def step(cur, b):
      nxt = cur + 1

      @pl.when(nxt < nch)
      def _():
        wait_idx(nxt, b ^ 1)
        fire_gather(nxt, b ^ 1)
      wait_gather(cur, b)
      compute_scatter(b)

      @pl.when(cur + 2 < nch)
      def _():
        fire_idx(cur + 2, b)

    fire_idx(0, 0)
    fire_idx(1, 1)
    wait_idx(0, 0)
    fire_gather(0, 0)

    def pair(i, _):
      step(2 * i, 0)
      step(2 * i + 1, 1)
      return 0
    lax.fori_loop(0, nch // 2, pair, 0)
    plsc.subcore_barrier()

    @pl.when(sid < NSTRIPE)
    def _():
      for b in range(rows_pt // zrows):
        r0 = pl.multiple_of(sid * rows_pt + b * zrows, 8)
        pltpu.sync_copy(agg_sh.at[pl.ds(r0, zrows), :],
                        out_hbm.at[cid, pl.ds(r0, zrows), :])

  return pl.kernel(
      body,
      out_type=jax.ShapeDtypeStruct((NC, n_nodes, D), f32),
      mesh=_mesh(),
      scratch_types=[
          pltpu.VMEM((C,), i32),
          pltpu.VMEM((C,), i32),
          pltpu.VMEM((C,), i32),
          pltpu.VMEM((C,), i32),
          pltpu.VMEM((C, D), f32),
          pltpu.VMEM((C, D), f32),
          pltpu.VMEM((C, D), f32),
          pltpu.VMEM((C, D), f32),
          pltpu.VMEM((zrows, D), f32),
          pltpu.VMEM_SHARED((n_nodes, D), f32),
          pltpu.SemaphoreType.DMA,
          pltpu.SemaphoreType.DMA,
          pltpu.SemaphoreType.DMA,
          pltpu.SemaphoreType.DMA,
          pltpu.SemaphoreType.DMA,
          pltpu.SemaphoreType.DMA,
      ],
  )(feat, ew, src, dst)


def _tc_pack(a3, attn_w2):
  A_, N, D = a3.shape
  blk = 1000

  def body(a_ref, w_ref, o_ref):
    a = a_ref[...]
    o_ref[:, 0:A_, :] = jnp.swapaxes(a, 0, 1)
    s = jnp.sum(a * w_ref[0], axis=-1)
    st = s.T
    row = jnp.concatenate(
        [st, st, st, st, jnp.zeros((blk, D - 4 * A_), f32)], axis=1)
    o_ref[:, A_:A_ + 1, :] = row[:, None, :]

  return pl.pallas_call(
      body,
      grid=(N // blk,),
      in_specs=[
          pl.BlockSpec((A_, blk, D), lambda i: (0, i, 0)),
          pl.BlockSpec((1, D), lambda i: (0, 0)),
      ],
      out_specs=pl.BlockSpec((blk, A_ + 1, D), lambda i: (i, 0, 0)),
      out_shape=jax.ShapeDtypeStruct((N, A_ + 1, D), f32),
  )(a3, attn_w2)


def _tc_feat1(center, odp):
  N, D = center.shape

  def body(c_ref, od_ref, o_ref):
    od = jnp.maximum(od_ref[:, 0:1] + od_ref[:, 1:2], 1.0)
    o_ref[...] = c_ref[...] * lax.rsqrt(od)

  return pl.pallas_call(
      body,
      out_shape=jax.ShapeDtypeStruct((N, D), f32),
  )(center, odp)


def _tc_dense(aggp, odp, idp, W, b2, g2, be2):
  _, N, D = aggp.shape

  def body(a_ref, od_ref, id_ref, w_ref, b_ref, g_ref, be_ref, o_ref):
    idn = lax.rsqrt(jnp.maximum(id_ref[:, 0:1] + id_ref[:, 1:2], 1.0))
    agg = (a_ref[0] + a_ref[1]) * idn
    y = jnp.dot(agg, w_ref[...], preferred_element_type=f32) + b_ref[...]
    mu = jnp.mean(y, axis=0, keepdims=True)
    var = jnp.mean((y - mu) ** 2, axis=0, keepdims=True)
    yn = (y - mu) * lax.rsqrt(var + 1e-5) * g_ref[...] + be_ref[...]
    yn = jnp.where(yn > 0, yn, jnp.exp(jnp.minimum(yn, 0.0)) - 1.0)
    odn = lax.rsqrt(jnp.maximum(od_ref[:, 0:1] + od_ref[:, 1:2], 1.0))
    o_ref[...] = yn * odn

  return pl.pallas_call(
      body,
      out_shape=jax.ShapeDtypeStruct((N, D), f32),
  )(aggp, odp, idp, W, b2, g2, be2)


def _tc_final(aggp, idp, W, b2, g2, be2, linW, linb2):
  _, N, D = aggp.shape
  NL = linW.shape[1]

  def body(a_ref, id_ref, w_ref, b_ref, g_ref, be_ref, lw_ref, lb_ref,
           h_ref, o2_ref):
    idn = lax.rsqrt(jnp.maximum(id_ref[:, 0:1] + id_ref[:, 1:2], 1.0))
    agg = (a_ref[0] + a_ref[1]) * idn
    y = jnp.dot(agg, w_ref[...], preferred_element_type=f32) + b_ref[...]
    mu = jnp.mean(y, axis=0, keepdims=True)
    var = jnp.mean((y - mu) ** 2, axis=0, keepdims=True)
    h = (y - mu) * lax.rsqrt(var + 1e-5) * g_ref[...] + be_ref[...]
    h_ref[...] = h
    o2_ref[...] = jnp.dot(h, lw_ref[...], preferred_element_type=f32) + lb_ref[...]

  return pl.pallas_call(
      body,
      out_shape=(
          jax.ShapeDtypeStruct((N, D), f32),
          jax.ShapeDtypeStruct((N, NL), f32),
      ),
  )(aggp, idp, W, b2, g2, be2, linW, linb2)


def kernel(edge_index, edge_w, gumbel_noise, aspect_emb, center_emb,
           attn_W, attn_b, W1, b1, W2, b2, W3, b3,
           bn_gamma, bn_beta, lin_W, lin_b):
  N, D = center_emb.shape
  A_ = gumbel_noise.shape[1]
  E = edge_index.shape[1]

  src = edge_index[0]
  dst = edge_index[1]
  w0 = edge_w[:, 0]
  w1 = edge_w[:, 1]
  a3 = aspect_emb.reshape(A_, N, D)
  gnflat = gumbel_noise.reshape(-1)

  degflat = _degrees(src, dst, N)
  NP = degflat.shape[0] // NC
  degout = degflat.reshape(NC, NP)
  odp = degout[:, :N].T
  idp = degout[:, N:2 * N].T

  aspTX = _tc_pack(a3, attn_W.reshape(1, D)).reshape(N, (A_ + 1) * D)
  ew = _edge_weight(w0, w1, gnflat, aspTX)

  feat = _tc_feat1(center_emb, odp)
  for (W, b) in ((W1, b1), (W2, b2)):
    aggp = _gconv_scatter(feat, ew, src, dst)
    feat = _tc_dense(aggp, odp, idp, W, b.reshape(1, D),
                     bn_gamma.reshape(1, D), bn_beta.reshape(1, D))
  aggp = _gconv_scatter(feat, ew, src, dst)
  h, out2 = _tc_final(aggp, idp, W3, b3.reshape(1, D),
                      bn_gamma.reshape(1, D), bn_beta.reshape(1, D),
                      lin_W, lin_b.reshape(1, lin_W.shape[1]))
  return (h, out2)


# speedup vs baseline: 8.8605x; 1.3055x over previous
import functools

import jax
import jax.numpy as jnp
from jax import lax
from jax.experimental import pallas as pl
from jax.experimental.pallas import tpu as pltpu
from jax.experimental.pallas import tpu_sc as plsc

f32 = jnp.float32
i32 = jnp.int32

NC = 2
NS = 16
NW = NC * NS
L = 16


def _perm(x, idx):
  dn = lax.GatherDimensionNumbers(
      offset_dims=(), collapsed_slice_dims=(0,), start_index_map=(0,))
  return lax.gather(x, idx[:, None], dn, slice_sizes=(1,),
                    mode=lax.GatherScatterMode.PROMISE_IN_BOUNDS)


def _mesh():
  return plsc.VectorSubcoreMesh(core_axis_name="c", subcore_axis_name="s")


def _degrees(src, dst, n_nodes):
  E = src.shape[0]
  ec = E // NW
  C = 80
  nch = ec // C
  NP = 2 * ((n_nodes + 639) // 640) * 640
  stripe = NP // NS

  def body(src_hbm, dst_hbm, out_hbm, sidx, didx, ones_v, zb, deg_sh):
    cid = lax.axis_index("c")
    sid = lax.axis_index("s")
    wid = sid * NC + cid

    def fill(i, _):
      ones_v[pl.ds(i * L, L)] = jnp.full((L,), 1.0, f32)
      return 0
    lax.fori_loop(0, C // L, fill, 0)

    def zfill(i, _):
      zb[pl.ds(i * L, L)] = jnp.zeros((L,), f32)
      return 0
    lax.fori_loop(0, stripe // L, zfill, 0)
    pltpu.sync_copy(zb, deg_sh.at[pl.ds(sid * stripe, stripe)])
    plsc.subcore_barrier()

    def chunk(ch, _):
      eb = wid * ec + ch * C
      pltpu.sync_copy(src_hbm.at[pl.ds(eb, C)], sidx)
      pltpu.sync_copy(dst_hbm.at[pl.ds(eb, C)], didx)

      def shift(q, _):
        didx[pl.ds(q * L, L)] = didx[pl.ds(q * L, L)] + n_nodes
        return 0
      lax.fori_loop(0, C // L, shift, 0)
      pltpu.sync_copy(ones_v, deg_sh.at[sidx], add=True)
      pltpu.sync_copy(ones_v, deg_sh.at[didx], add=True)
      return 0
    lax.fori_loop(0, nch, chunk, 0)
    plsc.subcore_barrier()
    o0 = pl.multiple_of(cid * NP + sid * stripe, 128)
    pltpu.sync_copy(deg_sh.at[pl.ds(pl.multiple_of(sid * stripe, 128), stripe)],
                    out_hbm.at[pl.ds(o0, stripe)])

  return pl.kernel(
      body,
      out_type=jax.ShapeDtypeStruct((NC * NP,), f32),
      mesh=_mesh(),
      scratch_types=[
          pltpu.VMEM((C,), i32),
          pltpu.VMEM((C,), i32),
          pltpu.VMEM((C,), f32),
          pltpu.VMEM((stripe,), f32),
          pltpu.VMEM_SHARED((NP,), f32),
      ],
  )(src, dst)


def _edge_weight(w0, w1, gnflat, aspTX):
  E = w0.shape[0]
  DM = aspTX.shape[1]
  D = 128
  ec = E // NW
  C = 40
  nch = ec // C
  npair = nch // 2

  def body(w0_hbm, w1_hbm, gn_hbm, asp_hbm, ew_hbm,
           w0va, w0vb, w1va, w1vb, gnva, gnvb,
           ab0a, ab0b, ab1a, ab1b, ewba, ewbb,
           semi0, semi1, semg0, semg1, semo0, semo1):
    w0v = (w0va, w0vb)
    w1v = (w1va, w1vb)
    gnv = (gnva, gnvb)
    ab0 = (ab0a, ab0b)
    ab1 = (ab1a, ab1b)
    ewb = (ewba, ewbb)
    cid = lax.axis_index("c")
    sid = lax.axis_index("s")
    wid = sid * NC + cid
    iot = lax.iota(i32, L)
    kv = iot & 3
    semi = (semi0, semi1)
    semg = (semg0, semg1)
    semo = (semo0, semo1)

    def ebase(g):
      return wid * ec + g * C

    def fire_idx(g, b):
      eb = ebase(g)
      pltpu.async_copy(w0_hbm.at[pl.ds(eb, C)], w0v[b], semi[b])
      pltpu.async_copy(w1_hbm.at[pl.ds(eb, C)], w1v[b], semi[b])
      pltpu.async_copy(gn_hbm.at[pl.ds(eb * 4, C * 4)], gnv[b], semi[b])

    def wait_idx(g, b):
      eb = ebase(g)
      pltpu.make_async_copy(w0_hbm.at[pl.ds(eb, C)], w0v[b], semi[b]).wait()
      pltpu.make_async_copy(w1_hbm.at[pl.ds(eb, C)], w1v[b], semi[b]).wait()
      pltpu.make_async_copy(gn_hbm.at[pl.ds(eb * 4, C * 4)], gnv[b], semi[b]).wait()

    def fire_gather(b):
      pltpu.async_copy(asp_hbm.at[w0v[b]], ab0[b], semg[b])
      pltpu.async_copy(asp_hbm.at[w1v[b]], ab1[b], semg[b])

    def wait_gather(b):
      pltpu.make_async_copy(asp_hbm.at[w0v[b]], ab0[b], semg[b]).wait()
      pltpu.make_async_copy(asp_hbm.at[w1v[b]], ab1[b], semg[b]).wait()

    def fire_store(g, b):
      eb = pl.multiple_of(ebase(g), 8)
      pltpu.async_copy(ewb[b], ew_hbm.at[pl.ds(eb, C), :], semo[b])

    def wait_store(g, b):
      eb = pl.multiple_of(ebase(g), 8)
      pltpu.make_async_copy(ewb[b], ew_hbm.at[pl.ds(eb, C), :], semo[b]).wait()

    def compute(b):
      def edge(je, _):
        s0 = ab0[b][je, pl.ds(4 * D, L)]
        s1 = ab1[b][je, pl.ds(4 * D, L)]
        gg = gnv[b][pl.ds((je >> 2) * L, L)]
        gn_e = _perm(gg, kv + 4 * (je & 3))
        t = (s0 + s1 + gn_e) * 2.0
        m = jnp.maximum(t, _perm(t, iot ^ 1))
        m = jnp.maximum(m, _perm(m, iot ^ 2))
        p = jnp.exp(t - m)
        q = p + _perm(p, iot ^ 1)
        q = q + _perm(q, iot ^ 2)
        attn = p / q
        accs = [None] * (D // L)
        for k in range(4):
          a = attn[k]
          for d in range(D // L):
            off = k * D + d * L
            term = ab0[b][je, pl.ds(off, L)] + ab1[b][je, pl.ds(off, L)]
            accs[d] = term * a if accs[d] is None else accs[d] + term * a
        for d in range(D // L):
          ewb[b][je, pl.ds(d * L, L)] = accs[d]
        return 0
      lax.fori_loop(0, C, edge, 0)

    def step(cur, b):
      nxt = cur + 1

      @pl.when(nxt < nch)
      def _():
        wait_idx(nxt, b ^ 1)
        fire_gather(b ^ 1)
      wait_gather(b)

      @pl.when(cur >= 2)
      def _():
        wait_store(cur, b)
      compute(b)
      fire_store(cur, b)

      @pl.when(cur + 2 < nch)
      def _():
        fire_idx(cur + 2, b)

    fire_idx(0, 0)
    fire_idx(1, 1)
    wait_idx(0, 0)
    fire_gather(0)

    def pair(i, _):
      step(2 * i, 0)
      step(2 * i + 1, 1)
      return 0
    lax.fori_loop(0, npair, pair, 0)
    wait_store(nch - 2, 0)
    wait_store(nch - 1, 1)

  return pl.kernel(
      body,
      out_type=jax.ShapeDtypeStruct((E, D), f32),
      mesh=_mesh(),
      scratch_types=[
          pltpu.VMEM((C,), i32),
          pltpu.VMEM((C,), i32),
          pltpu.VMEM((C,), i32),
          pltpu.VMEM((C,), i32),
          pltpu.VMEM((4 * C,), f32),
          pltpu.VMEM((4 * C,), f32),
          pltpu.VMEM((C, DM), f32),
          pltpu.VMEM((C, DM), f32),
          pltpu.VMEM((C, DM), f32),
          pltpu.VMEM((C, DM), f32),
          pltpu.VMEM((C, D), f32),
          pltpu.VMEM((C, D), f32),
          pltpu.SemaphoreType.DMA,
          pltpu.SemaphoreType.DMA,
          pltpu.SemaphoreType.DMA,
          pltpu.SemaphoreType.DMA,
          pltpu.SemaphoreType.DMA,
          pltpu.SemaphoreType.DMA,
      ],
  )(w0, w1, gnflat, aspTX)


def _gconv_scatter(feat, ew, src, dst):
  n_nodes, D = feat.shape
  E = src.shape[0]
  ec = E // NW
  C = 40
  nch = ec // C
  NSTRIPE = 10
  rows_pt = n_nodes // NSTRIPE
  zrows = 40

  def body(feat_hbm, ew_hbm, src_hbm, dst_hbm, out_hbm,
           sidxa, sidxb, didxa, didxb, fva, fvb, eva, evb, zb, agg_sh,
           semi0, semi1, semg0, semg1, seme0, seme1):
    sidx = (sidxa, sidxb)
    didx = (didxa, didxb)
    fv = (fva, fvb)
    ev = (eva, evb)
    cid = lax.axis_index("c")
    sid = lax.axis_index("s")
    wid = sid * NC + cid

    def zfill(j, _):
      for d in range(D // L):
        zb[j, pl.ds(d * L, L)] = jnp.zeros((L,), f32)
      return 0
    lax.fori_loop(0, zrows, zfill, 0)

    @pl.when(sid < NSTRIPE)
    def _():
      for b in range(rows_pt // zrows):
        r0 = pl.multiple_of(sid * rows_pt + b * zrows, 8)
        pltpu.sync_copy(zb, agg_sh.at[pl.ds(r0, zrows), :])
    plsc.subcore_barrier()

    semi = (semi0, semi1)
    semg = (semg0, semg1)
    seme = (seme0, seme1)

    def ebase(g):
      return wid * ec + g * C

    def fire_idx(g, b):
      eb = ebase(g)
      pltpu.async_copy(src_hbm.at[pl.ds(eb, C)], sidx[b], semi[b])
      pltpu.async_copy(dst_hbm.at[pl.ds(eb, C)], didx[b], semi[b])

    def wait_idx(g, b):
      eb = ebase(g)
      pltpu.make_async_copy(src_hbm.at[pl.ds(eb, C)], sidx[b], semi[b]).wait()
      pltpu.make_async_copy(dst_hbm.at[pl.ds(eb, C)], didx[b], semi[b]).wait()

    def fire_gather(g, b):
      eb = pl.multiple_of(ebase(g), 8)
      pltpu.async_copy(feat_hbm.at[sidx[b]], fv[b], semg[b])
      pltpu.async_copy(ew_hbm.at[pl.ds(eb, C), :], ev[b], seme[b])

    def wait_gather(g, b):
      eb = pl.multiple_of(ebase(g), 8)
      pltpu.make_async_copy(feat_hbm.at[sidx[b]], fv[b], semg[b]).wait()
      pltpu.make_async_copy(ew_hbm.at[pl.ds(eb, C), :], ev[b], seme[b]).wait()

    def compute_scatter(b):
      def edge(je, _):
        for d in range(D // L):
          fv[b][je, pl.ds(d * L, L)] = (
              fv[b][je, pl.ds(d * L, L)] * ev[b][je, pl.ds(d * L, L)])
        return 0
      lax.fori_loop(0, C, edge, 0)
      pltpu.sync_copy(fv[b], agg_sh.at[didx[b]], add=True)

    def step(cur, b):
      nxt = cur + 1

      @pl.when(nxt < nch)
      def _():
        wait_idx(nxt, b ^ 1)
        fire_gather(nxt, b ^ 1)
      wait_gather(cur, b)
      compute_scatter(b)

      @pl.when(cur + 2 < nch)
      def _():
        fire_idx(cur + 2, b)

    fire_idx(0, 0)
    fire_idx(1, 1)
    wait_idx(0, 0)
    fire_gather(0, 0)

    def pair(i, _):
      step(2 * i, 0)
      step(2 * i + 1, 1)
      return 0
    lax.fori_loop(0, nch // 2, pair, 0)
    plsc.subcore_barrier()

    @pl.when(sid < NSTRIPE)
    def _():
      for b in range(rows_pt // zrows):
        r0 = pl.multiple_of(sid * rows_pt + b * zrows, 8)
        pltpu.sync_copy(agg_sh.at[pl.ds(r0, zrows), :],
                        out_hbm.at[cid, pl.ds(r0, zrows), :])

  return pl.kernel(
      body,
      out_type=jax.ShapeDtypeStruct((NC, n_nodes, D), f32),
      mesh=_mesh(),
      scratch_types=[
          pltpu.VMEM((C,), i32),
          pltpu.VMEM((C,), i32),
          pltpu.VMEM((C,), i32),
          pltpu.VMEM((C,), i32),
          pltpu.VMEM((C, D), f32),
          pltpu.VMEM((C, D), f32),
          pltpu.VMEM((C, D), f32),
          pltpu.VMEM((C, D), f32),
          pltpu.VMEM((zrows, D), f32),
          pltpu.VMEM_SHARED((n_nodes, D), f32),
          pltpu.SemaphoreType.DMA,
          pltpu.SemaphoreType.DMA,
          pltpu.SemaphoreType.DMA,
          pltpu.SemaphoreType.DMA,
          pltpu.SemaphoreType.DMA,
          pltpu.SemaphoreType.DMA,
      ],
  )(feat, ew, src, dst)


def _tc_pack(a3, attn_w2):
  A_, N, D = a3.shape
  blk = 1000

  def body(a_ref, w_ref, o_ref):
    a = a_ref[...]
    o_ref[:, 0:A_, :] = jnp.swapaxes(a, 0, 1)
    s = jnp.sum(a * w_ref[0], axis=-1)
    st = s.T
    row = jnp.concatenate(
        [st, st, st, st, jnp.zeros((blk, D - 4 * A_), f32)], axis=1)
    o_ref[:, A_:A_ + 1, :] = row[:, None, :]

  return pl.pallas_call(
      body,
      grid=(N // blk,),
      in_specs=[
          pl.BlockSpec((A_, blk, D), lambda i: (0, i, 0)),
          pl.BlockSpec((1, D), lambda i: (0, 0)),
      ],
      out_specs=pl.BlockSpec((blk, A_ + 1, D), lambda i: (i, 0, 0)),
      out_shape=jax.ShapeDtypeStruct((N, A_ + 1, D), f32),
  )(a3, attn_w2)


def _tc_feat1(center, odp):
  N, D = center.shape

  def body(c_ref, od_ref, o_ref):
    od = jnp.maximum(od_ref[:, 0:1] + od_ref[:, 1:2], 1.0)
    o_ref[...] = c_ref[...] * lax.rsqrt(od)

  return pl.pallas_call(
      body,
      out_shape=jax.ShapeDtypeStruct((N, D), f32),
  )(center, odp)


def _tc_dense(aggp, odp, idp, W, b2, g2, be2):
  _, N, D = aggp.shape

  def body(a_ref, od_ref, id_ref, w_ref, b_ref, g_ref, be_ref, o_ref):
    idn = lax.rsqrt(jnp.maximum(id_ref[:, 0:1] + id_ref[:, 1:2], 1.0))
    agg = (a_ref[0] + a_ref[1]) * idn
    y = jnp.dot(agg, w_ref[...], preferred_element_type=f32) + b_ref[...]
    mu = jnp.mean(y, axis=0, keepdims=True)
    var = jnp.mean((y - mu) ** 2, axis=0, keepdims=True)
    yn = (y - mu) * lax.rsqrt(var + 1e-5) * g_ref[...] + be_ref[...]
    yn = jnp.where(yn > 0, yn, jnp.exp(jnp.minimum(yn, 0.0)) - 1.0)
    odn = lax.rsqrt(jnp.maximum(od_ref[:, 0:1] + od_ref[:, 1:2], 1.0))
    o_ref[...] = yn * odn

  return pl.pallas_call(
      body,
      out_shape=jax.ShapeDtypeStruct((N, D), f32),
  )(aggp, odp, idp, W, b2, g2, be2)


def _tc_final(aggp, idp, W, b2, g2, be2, linW, linb2):
  _, N, D = aggp.shape
  NL = linW.shape[1]

  def body(a_ref, id_ref, w_ref, b_ref, g_ref, be_ref, lw_ref, lb_ref,
           h_ref, o2_ref):
    idn = lax.rsqrt(jnp.maximum(id_ref[:, 0:1] + id_ref[:, 1:2], 1.0))
    agg = (a_ref[0] + a_ref[1]) * idn
    y = jnp.dot(agg, w_ref[...], preferred_element_type=f32) + b_ref[...]
    mu = jnp.mean(y, axis=0, keepdims=True)
    var = jnp.mean((y - mu) ** 2, axis=0, keepdims=True)
    h = (y - mu) * lax.rsqrt(var + 1e-5) * g_ref[...] + be_ref[...]
    h_ref[...] = h
    o2_ref[...] = jnp.dot(h, lw_ref[...], preferred_element_type=f32) + lb_ref[...]

  return pl.pallas_call(
      body,
      out_shape=(
          jax.ShapeDtypeStruct((N, D), f32),
          jax.ShapeDtypeStruct((N, NL), f32),
      ),
  )(aggp, idp, W, b2, g2, be2, linW, linb2)


def kernel(edge_index, edge_w, gumbel_noise, aspect_emb, center_emb,
           attn_W, attn_b, W1, b1, W2, b2, W3, b3,
           bn_gamma, bn_beta, lin_W, lin_b):
  N, D = center_emb.shape
  A_ = gumbel_noise.shape[1]
  E = edge_index.shape[1]

  src = edge_index[0]
  dst = edge_index[1]
  w0 = edge_w[:, 0]
  w1 = edge_w[:, 1]
  a3 = aspect_emb.reshape(A_, N, D)
  gnflat = gumbel_noise.reshape(-1)

  degflat = _degrees(src, dst, N)
  NP = degflat.shape[0] // NC
  degout = degflat.reshape(NC, NP)
  odp = degout[:, :N].T
  idp = degout[:, N:2 * N].T

  aspTX = _tc_pack(a3, attn_W.reshape(1, D)).reshape(N, (A_ + 1) * D)
  ew = _edge_weight(w0, w1, gnflat, aspTX)

  feat = _tc_feat1(center_emb, odp)
  for (W, b) in ((W1, b1), (W2, b2)):
    aggp = _gconv_scatter(feat, ew, src, dst)
    feat = _tc_dense(aggp, odp, idp, W, b.reshape(1, D),
                     bn_gamma.reshape(1, D), bn_beta.reshape(1, D))
  aggp = _gconv_scatter(feat, ew, src, dst)
  h, out2 = _tc_final(aggp, idp, W3, b3.reshape(1, D),
                      bn_gamma.reshape(1, D), bn_beta.reshape(1, D),
                      lin_W, lin_b.reshape(1, lin_W.shape[1]))
  return (h, out2)


# speedup vs baseline: 8.8704x; 1.0011x over previous
import functools

import jax
import jax.numpy as jnp
from jax import lax
from jax.experimental import pallas as pl
from jax.experimental.pallas import tpu as pltpu
from jax.experimental.pallas import tpu_sc as plsc

f32 = jnp.float32
i32 = jnp.int32

NC = 2
NS = 16
NW = NC * NS
L = 16


def _perm(x, idx):
  dn = lax.GatherDimensionNumbers(
      offset_dims=(), collapsed_slice_dims=(0,), start_index_map=(0,))
  return lax.gather(x, idx[:, None], dn, slice_sizes=(1,),
                    mode=lax.GatherScatterMode.PROMISE_IN_BOUNDS)


def _mesh():
  return plsc.VectorSubcoreMesh(core_axis_name="c", subcore_axis_name="s")


def _degrees(src, dst, n_nodes):
  E = src.shape[0]
  ec = E // NW
  C = 80
  nch = ec // C
  HP = ((n_nodes + 639) // 640) * 640
  NP = 2 * HP
  stripe = HP // NS

  def body(src_hbm, dst_hbm, out_hbm, sidxa, sidxb, didxa, didxb,
           ones_v, zb, degO, degI, semi0, semi1, sems0, sems1):
    cid = lax.axis_index("c")
    sid = lax.axis_index("s")
    wid = sid * NC + cid
    sidx = (sidxa, sidxb)
    didx = (didxa, didxb)
    semi = (semi0, semi1)
    sems = (sems0, sems1)

    def fill(i, _):
      ones_v[pl.ds(i * L, L)] = jnp.full((L,), 1.0, f32)
      return 0
    lax.fori_loop(0, C // L, fill, 0)

    def zfill(i, _):
      zb[pl.ds(i * L, L)] = jnp.zeros((L,), f32)
      return 0
    lax.fori_loop(0, stripe // L, zfill, 0)
    z0 = pl.multiple_of(sid * stripe, 128)
    pltpu.sync_copy(zb, degO.at[pl.ds(z0, stripe)])
    pltpu.sync_copy(zb, degI.at[pl.ds(z0, stripe)])
    plsc.subcore_barrier()

    def ebase(g):
      return wid * ec + g * C

    def fire_idx(g, b):
      eb = ebase(g)
      pltpu.async_copy(src_hbm.at[pl.ds(eb, C)], sidx[b], semi[b])
      pltpu.async_copy(dst_hbm.at[pl.ds(eb, C)], didx[b], semi[b])

    def wait_idx(g, b):
      eb = ebase(g)
      pltpu.make_async_copy(src_hbm.at[pl.ds(eb, C)], sidx[b], semi[b]).wait()
      pltpu.make_async_copy(dst_hbm.at[pl.ds(eb, C)], didx[b], semi[b]).wait()

    def fire_scat(b):
      pltpu.async_copy(ones_v, degO.at[sidx[b]], sems[b], add=True)
      pltpu.async_copy(ones_v, degI.at[didx[b]], sems[b], add=True)

    def wait_scat(b):
      pltpu.make_async_copy(ones_v, degO.at[sidx[b]], sems[b]).wait()
      pltpu.make_async_copy(ones_v, degI.at[didx[b]], sems[b]).wait()

    def step(cur, b):
      wait_idx(cur, b)

      @pl.when(cur >= 2)
      def _():
        wait_scat(b)
      fire_scat(b)

      @pl.when(cur + 2 < nch)
      def _():
        fire_idx(cur + 2, b)

    fire_idx(0, 0)
    fire_idx(1, 1)

    def pair(i, _):
      step(2 * i, 0)
      step(2 * i + 1, 1)
      return 0
    lax.fori_loop(0, nch // 2, pair, 0)

    if nch % 2 == 1:
      step(nch - 1, 0)
    wait_scat(nch % 2)
    wait_scat((nch + 1) % 2)
    plsc.subcore_barrier()
    o0 = pl.multiple_of(cid * NP + sid * stripe, 128)
    o1 = pl.multiple_of(cid * NP + HP + sid * stripe, 128)
    pltpu.sync_copy(degO.at[pl.ds(z0, stripe)], out_hbm.at[pl.ds(o0, stripe)])
    pltpu.sync_copy(degI.at[pl.ds(z0, stripe)], out_hbm.at[pl.ds(o1, stripe)])

  return pl.kernel(
      body,
      out_type=jax.ShapeDtypeStruct((NC * NP,), f32),
      mesh=_mesh(),
      scratch_types=[
          pltpu.VMEM((C,), i32),
          pltpu.VMEM((C,), i32),
          pltpu.VMEM((C,), i32),
          pltpu.VMEM((C,), i32),
          pltpu.VMEM((C,), f32),
          pltpu.VMEM((stripe,), f32),
          pltpu.VMEM_SHARED((HP,), f32),
          pltpu.VMEM_SHARED((HP,), f32),
          pltpu.SemaphoreType.DMA,
          pltpu.SemaphoreType.DMA,
          pltpu.SemaphoreType.DMA,
          pltpu.SemaphoreType.DMA,
      ],
  )(src, dst)


def _edge_weight(w0, w1, gnflat, aspTX):
  E = w0.shape[0]
  DM = aspTX.shape[1]
  D = 128
  ec = E // NW
  C = 40
  nch = ec // C
  npair = nch // 2

  def body(w0_hbm, w1_hbm, gn_hbm, asp_hbm, ew_hbm,
           w0va, w0vb, w1va, w1vb, gnva, gnvb,
           ab0a, ab0b, ab1a, ab1b, ewba, ewbb,
           semi0, semi1, semg0, semg1, semo0, semo1):
    w0v = (w0va, w0vb)
    w1v = (w1va, w1vb)
    gnv = (gnva, gnvb)
    ab0 = (ab0a, ab0b)
    ab1 = (ab1a, ab1b)
    ewb = (ewba, ewbb)
    cid = lax.axis_index("c")
    sid = lax.axis_index("s")
    wid = sid * NC + cid
    iot = lax.iota(i32, L)
    kv = iot & 3
    semi = (semi0, semi1)
    semg = (semg0, semg1)
    semo = (semo0, semo1)

    def ebase(g):
      return wid * ec + g * C

    def fire_idx(g, b):
      eb = ebase(g)
      pltpu.async_copy(w0_hbm.at[pl.ds(eb, C)], w0v[b], semi[b])
      pltpu.async_copy(w1_hbm.at[pl.ds(eb, C)], w1v[b], semi[b])
      pltpu.async_copy(gn_hbm.at[pl.ds(eb * 4, C * 4)], gnv[b], semi[b])

    def wait_idx(g, b):
      eb = ebase(g)
      pltpu.make_async_copy(w0_hbm.at[pl.ds(eb, C)], w0v[b], semi[b]).wait()
      pltpu.make_async_copy(w1_hbm.at[pl.ds(eb, C)], w1v[b], semi[b]).wait()
      pltpu.make_async_copy(gn_hbm.at[pl.ds(eb * 4, C * 4)], gnv[b], semi[b]).wait()

    def fire_gather(b):
      pltpu.async_copy(asp_hbm.at[w0v[b]], ab0[b], semg[b])
      pltpu.async_copy(asp_hbm.at[w1v[b]], ab1[b], semg[b])

    def wait_gather(b):
      pltpu.make_async_copy(asp_hbm.at[w0v[b]], ab0[b], semg[b]).wait()
      pltpu.make_async_copy(asp_hbm.at[w1v[b]], ab1[b], semg[b]).wait()

    def fire_store(g, b):
      eb = pl.multiple_of(ebase(g), 8)
      pltpu.async_copy(ewb[b], ew_hbm.at[pl.ds(eb, C), :], semo[b])

    def wait_store(g, b):
      eb = pl.multiple_of(ebase(g), 8)
      pltpu.make_async_copy(ewb[b], ew_hbm.at[pl.ds(eb, C), :], semo[b]).wait()

    def compute(b):
      def edge(je, _):
        s0 = ab0[b][je, pl.ds(4 * D, L)]
        s1 = ab1[b][je, pl.ds(4 * D, L)]
        gg = gnv[b][pl.ds((je >> 2) * L, L)]
        gn_e = _perm(gg, kv + 4 * (je & 3))
        t = (s0 + s1 + gn_e) * 2.0
        m = jnp.maximum(t, _perm(t, iot ^ 1))
        m = jnp.maximum(m, _perm(m, iot ^ 2))
        p = jnp.exp(t - m)
        q = p + _perm(p, iot ^ 1)
        q = q + _perm(q, iot ^ 2)
        attn = p / q
        accs = [None] * (D // L)
        for k in range(4):
          a = attn[k]
          for d in range(D // L):
            off = k * D + d * L
            term = ab0[b][je, pl.ds(off, L)] + ab1[b][je, pl.ds(off, L)]
            accs[d] = term * a if accs[d] is None else accs[d] + term * a
        for d in range(D // L):
          ewb[b][je, pl.ds(d * L, L)] = accs[d]
        return 0
      lax.fori_loop(0, C, edge, 0)

    def step(cur, b):
      nxt = cur + 1

      @pl.when(nxt < nch)
      def _():
        wait_idx(nxt, b ^ 1)
        fire_gather(b ^ 1)
      wait_gather(b)

      @pl.when(cur >= 2)
      def _():
        wait_store(cur, b)
      compute(b)
      fire_store(cur, b)

      @pl.when(cur + 2 < nch)
      def _():
        fire_idx(cur + 2, b)

    fire_idx(0, 0)
    fire_idx(1, 1)
    wait_idx(0, 0)
    fire_gather(0)

    def pair(i, _):
      step(2 * i, 0)
      step(2 * i + 1, 1)
      return 0
    lax.fori_loop(0, npair, pair, 0)
    wait_store(nch - 2, 0)
    wait_store(nch - 1, 1)

  return pl.kernel(
      body,
      out_type=jax.ShapeDtypeStruct((E, D), f32),
      mesh=_mesh(),
      scratch_types=[
          pltpu.VMEM((C,), i32),
          pltpu.VMEM((C,), i32),
          pltpu.VMEM((C,), i32),
          pltpu.VMEM((C,), i32),
          pltpu.VMEM((4 * C,), f32),
          pltpu.VMEM((4 * C,), f32),
          pltpu.VMEM((C, DM), f32),
          pltpu.VMEM((C, DM), f32),
          pltpu.VMEM((C, DM), f32),
          pltpu.VMEM((C, DM), f32),
          pltpu.VMEM((C, D), f32),
          pltpu.VMEM((C, D), f32),
          pltpu.SemaphoreType.DMA,
          pltpu.SemaphoreType.DMA,
          pltpu.SemaphoreType.DMA,
          pltpu.SemaphoreType.DMA,
          pltpu.SemaphoreType.DMA,
          pltpu.SemaphoreType.DMA,
      ],
  )(w0, w1, gnflat, aspTX)


def _gconv_scatter(feat, ew, src, dst):
  n_nodes, D = feat.shape
  E = src.shape[0]
  ec = E // NW
  C = 40
  nch = ec // C
  NSTRIPE = 10
  rows_pt = n_nodes // NSTRIPE
  zrows = 40

  def body(feat_hbm, ew_hbm, src_hbm, dst_hbm, out_hbm,
           sidxa, sidxb, didxa, didxb, fva, fvb, eva, evb, zb, agg_sh,
           semi0, semi1, semg0, semg1, seme0, seme1):
    sidx = (sidxa, sidxb)
    didx = (didxa, didxb)
    fv = (fva, fvb)
    ev = (eva, evb)
    cid = lax.axis_index("c")
    sid = lax.axis_index("s")
    wid = sid * NC + cid

    def zfill(j, _):
      for d in range(D // L):
        zb[j, pl.ds(d * L, L)] = jnp.zeros((L,), f32)
      return 0
    lax.fori_loop(0, zrows, zfill, 0)

    @pl.when(sid < NSTRIPE)
    def _():
      for b in range(rows_pt // zrows):
        r0 = pl.multiple_of(sid * rows_pt + b * zrows, 8)
        pltpu.sync_copy(zb, agg_sh.at[pl.ds(r0, zrows), :])
    plsc.subcore_barrier()

    semi = (semi0, semi1)
    semg = (semg0, semg1)
    seme = (seme0, seme1)

    def ebase(g):
      return wid * ec + g * C

    def fire_idx(g, b):
      eb = ebase(g)
      pltpu.async_copy(src_hbm.at[pl.ds(eb, C)], sidx[b], semi[b])
      pltpu.async_copy(dst_hbm.at[pl.ds(eb, C)], didx[b], semi[b])

    def wait_idx(g, b):
      eb = ebase(g)
      pltpu.make_async_copy(src_hbm.at[pl.ds(eb, C)], sidx[b], semi[b]).wait()
      pltpu.make_async_copy(dst_hbm.at[pl.ds(eb, C)], didx[b], semi[b]).wait()

    def fire_gather(g, b):
      eb = pl.multiple_of(ebase(g), 8)
      pltpu.async_copy(feat_hbm.at[sidx[b]], fv[b], semg[b])
      pltpu.async_copy(ew_hbm.at[pl.ds(eb, C), :], ev[b], seme[b])

    def wait_gather(g, b):
      eb = pl.multiple_of(ebase(g), 8)
      pltpu.make_async_copy(feat_hbm.at[sidx[b]], fv[b], semg[b]).wait()
      pltpu.make_async_copy(ew_hbm.at[pl.ds(eb, C), :], ev[b], seme[b]).wait()

    def compute_scatter(b):
      def edge(jj, _):
        for u in range(2):
          je = 2 * jj + u
          for d in range(D // L):
            fv[b][je, pl.ds(d * L, L)] = (
                fv[b][je, pl.ds(d * L, L)] * ev[b][je, pl.ds(d * L, L)])
        return 0
      lax.fori_loop(0, C // 2, edge, 0)
      pltpu.sync_copy(fv[b], agg_sh.at[didx[b]], add=True)

    def step(cur, b):
      nxt = cur + 1

      @pl.when(nxt < nch)
      def _():
        wait_idx(nxt, b ^ 1)
        fire_gather(nxt, b ^ 1)
      wait_gather(cur, b)
      compute_scatter(b)

      @pl.when(cur + 2 < nch)
      def _():
        fire_idx(cur + 2, b)

    fire_idx(0, 0)
    fire_idx(1, 1)
    wait_idx(0, 0)
    fire_gather(0, 0)

    def pair(i, _):
      step(2 * i, 0)
      step(2 * i + 1, 1)
      return 0
    lax.fori_loop(0, nch // 2, pair, 0)
    plsc.subcore_barrier()

    @pl.when(sid < NSTRIPE)
    def _():
      for b in range(rows_pt // zrows):
        r0 = pl.multiple_of(sid * rows_pt + b * zrows, 8)
        pltpu.sync_copy(agg_sh.at[pl.ds(r0, zrows), :],
                        out_hbm.at[cid, pl.ds(r0, zrows), :])

  return pl.kernel(
      body,
      out_type=jax.ShapeDtypeStruct((NC, n_nodes, D), f32),
      mesh=_mesh(),
      scratch_types=[
          pltpu.VMEM((C,), i32),
          pltpu.VMEM((C,), i32),
          pltpu.VMEM((C,), i32),
          pltpu.VMEM((C,), i32),
          pltpu.VMEM((C, D), f32),
          pltpu.VMEM((C, D), f32),
          pltpu.VMEM((C, D), f32),
          pltpu.VMEM((C, D), f32),
          pltpu.VMEM((zrows, D), f32),
          pltpu.VMEM_SHARED((n_nodes, D), f32),
          pltpu.SemaphoreType.DMA,
          pltpu.SemaphoreType.DMA,
          pltpu.SemaphoreType.DMA,
          pltpu.SemaphoreType.DMA,
          pltpu.SemaphoreType.DMA,
          pltpu.SemaphoreType.DMA,
      ],
  )(feat, ew, src, dst)


def _tc_pack(a3, attn_w2):
  A_, N, D = a3.shape
  blk = 1000

  def body(a_ref, w_ref, o_ref):
    a = a_ref[...]
    o_ref[:, 0:A_, :] = jnp.swapaxes(a, 0, 1)
    s = jnp.sum(a * w_ref[0], axis=-1)
    st = s.T
    row = jnp.concatenate(
        [st, st, st, st, jnp.zeros((blk, D - 4 * A_), f32)], axis=1)
    o_ref[:, A_:A_ + 1, :] = row[:, None, :]

  return pl.pallas_call(
      body,
      grid=(N // blk,),
      in_specs=[
          pl.BlockSpec((A_, blk, D), lambda i: (0, i, 0)),
          pl.BlockSpec((1, D), lambda i: (0, 0)),
      ],
      out_specs=pl.BlockSpec((blk, A_ + 1, D), lambda i: (i, 0, 0)),
      out_shape=jax.ShapeDtypeStruct((N, A_ + 1, D), f32),
  )(a3, attn_w2)


def _tc_feat1(center, odp):
  N, D = center.shape

  def body(c_ref, od_ref, o_ref):
    od = jnp.maximum(od_ref[:, 0:1] + od_ref[:, 1:2], 1.0)
    o_ref[...] = c_ref[...] * lax.rsqrt(od)

  return pl.pallas_call(
      body,
      out_shape=jax.ShapeDtypeStruct((N, D), f32),
  )(center, odp)


def _tc_dense(aggp, odp, idp, W, b2, g2, be2):
  _, N, D = aggp.shape

  def body(a_ref, od_ref, id_ref, w_ref, b_ref, g_ref, be_ref, o_ref):
    idn = lax.rsqrt(jnp.maximum(id_ref[:, 0:1] + id_ref[:, 1:2], 1.0))
    agg = (a_ref[0] + a_ref[1]) * idn
    y = jnp.dot(agg, w_ref[...], preferred_element_type=f32) + b_ref[...]
    mu = jnp.mean(y, axis=0, keepdims=True)
    var = jnp.mean((y - mu) ** 2, axis=0, keepdims=True)
    yn = (y - mu) * lax.rsqrt(var + 1e-5) * g_ref[...] + be_ref[...]
    yn = jnp.where(yn > 0, yn, jnp.exp(jnp.minimum(yn, 0.0)) - 1.0)
    odn = lax.rsqrt(jnp.maximum(od_ref[:, 0:1] + od_ref[:, 1:2], 1.0))
    o_ref[...] = yn * odn

  return pl.pallas_call(
      body,
      out_shape=jax.ShapeDtypeStruct((N, D), f32),
  )(aggp, odp, idp, W, b2, g2, be2)


def _tc_final(aggp, idp, W, b2, g2, be2, linW, linb2):
  _, N, D = aggp.shape
  NL = linW.shape[1]

  def body(a_ref, id_ref, w_ref, b_ref, g_ref, be_ref, lw_ref, lb_ref,
           h_ref, o2_ref):
    idn = lax.rsqrt(jnp.maximum(id_ref[:, 0:1] + id_ref[:, 1:2], 1.0))
    agg = (a_ref[0] + a_ref[1]) * idn
    y = jnp.dot(agg, w_ref[...], preferred_element_type=f32) + b_ref[...]
    mu = jnp.mean(y, axis=0, keepdims=True)
    var = jnp.mean((y - mu) ** 2, axis=0, keepdims=True)
    h = (y - mu) * lax.rsqrt(var + 1e-5) * g_ref[...] + be_ref[...]
    h_ref[...] = h
    o2_ref[...] = jnp.dot(h, lw_ref[...], preferred_element_type=f32) + lb_ref[...]

  return pl.pallas_call(
      body,
      out_shape=(
          jax.ShapeDtypeStruct((N, D), f32),
          jax.ShapeDtypeStruct((N, NL), f32),
      ),
  )(aggp, idp, W, b2, g2, be2, linW, linb2)


def kernel(edge_index, edge_w, gumbel_noise, aspect_emb, center_emb,
           attn_W, attn_b, W1, b1, W2, b2, W3, b3,
           bn_gamma, bn_beta, lin_W, lin_b):
  N, D = center_emb.shape
  A_ = gumbel_noise.shape[1]
  E = edge_index.shape[1]

  src = edge_index[0]
  dst = edge_index[1]
  w0 = edge_w[:, 0]
  w1 = edge_w[:, 1]
  a3 = aspect_emb.reshape(A_, N, D)
  gnflat = gumbel_noise.reshape(-1)

  degflat = _degrees(src, dst, N)
  NP = degflat.shape[0] // NC
  HP = NP // 2
  degout = degflat.reshape(NC, NP)
  odp = degout[:, :N].T
  idp = degout[:, HP:HP + N].T

  aspTX = _tc_pack(a3, attn_W.reshape(1, D)).reshape(N, (A_ + 1) * D)
  ew = _edge_weight(w0, w1, gnflat, aspTX)

  feat = _tc_feat1(center_emb, odp)
  for (W, b) in ((W1, b1), (W2, b2)):
    aggp = _gconv_scatter(feat, ew, src, dst)
    feat = _tc_dense(aggp, odp, idp, W, b.reshape(1, D),
                     bn_gamma.reshape(1, D), bn_beta.reshape(1, D))
  aggp = _gconv_scatter(feat, ew, src, dst)
  h, out2 = _tc_final(aggp, idp, W3, b3.reshape(1, D),
                      bn_gamma.reshape(1, D), bn_beta.reshape(1, D),
                      lin_W, lin_b.reshape(1, lin_W.shape[1]))
  return (h, out2)


# speedup vs baseline: 9.3794x; 1.0574x over previous
import functools

import jax
import jax.numpy as jnp
from jax import lax
from jax.experimental import pallas as pl
from jax.experimental.pallas import tpu as pltpu
from jax.experimental.pallas import tpu_sc as plsc

f32 = jnp.float32
i32 = jnp.int32

NC = 2
NS = 16
NW = NC * NS
L = 16


def _perm(x, idx):
  dn = lax.GatherDimensionNumbers(
      offset_dims=(), collapsed_slice_dims=(0,), start_index_map=(0,))
  return lax.gather(x, idx[:, None], dn, slice_sizes=(1,),
                    mode=lax.GatherScatterMode.PROMISE_IN_BOUNDS)


def _mesh():
  return plsc.VectorSubcoreMesh(core_axis_name="c", subcore_axis_name="s")


def _degrees(src, dst, n_nodes):
  E = src.shape[0]
  ec = E // NW
  C = 80
  nch = ec // C
  HP = ((n_nodes + 639) // 640) * 640
  NP = 2 * HP
  stripe = HP // NS

  def body(src_hbm, dst_hbm, out_hbm, sidxa, sidxb, didxa, didxb,
           ones_v, zb, degO, degI, semi0, semi1, sems0, sems1):
    cid = lax.axis_index("c")
    sid = lax.axis_index("s")
    wid = sid * NC + cid
    sidx = (sidxa, sidxb)
    didx = (didxa, didxb)
    semi = (semi0, semi1)
    sems = (sems0, sems1)

    def fill(i, _):
      ones_v[pl.ds(i * L, L)] = jnp.full((L,), 1.0, f32)
      return 0
    lax.fori_loop(0, C // L, fill, 0)

    def zfill(i, _):
      zb[pl.ds(i * L, L)] = jnp.zeros((L,), f32)
      return 0
    lax.fori_loop(0, stripe // L, zfill, 0)
    z0 = pl.multiple_of(sid * stripe, 128)
    pltpu.sync_copy(zb, degO.at[pl.ds(z0, stripe)])
    pltpu.sync_copy(zb, degI.at[pl.ds(z0, stripe)])
    plsc.subcore_barrier()

    def ebase(g):
      return wid * ec + g * C

    def fire_idx(g, b):
      eb = ebase(g)
      pltpu.async_copy(src_hbm.at[pl.ds(eb, C)], sidx[b], semi[b])
      pltpu.async_copy(dst_hbm.at[pl.ds(eb, C)], didx[b], semi[b])

    def wait_idx(g, b):
      eb = ebase(g)
      pltpu.make_async_copy(src_hbm.at[pl.ds(eb, C)], sidx[b], semi[b]).wait()
      pltpu.make_async_copy(dst_hbm.at[pl.ds(eb, C)], didx[b], semi[b]).wait()

    def fire_scat(b):
      pltpu.async_copy(ones_v, degO.at[sidx[b]], sems[b], add=True)
      pltpu.async_copy(ones_v, degI.at[didx[b]], sems[b], add=True)

    def wait_scat(b):
      pltpu.make_async_copy(ones_v, degO.at[sidx[b]], sems[b]).wait()
      pltpu.make_async_copy(ones_v, degI.at[didx[b]], sems[b]).wait()

    def step(cur, b):
      wait_idx(cur, b)

      @pl.when(cur >= 2)
      def _():
        wait_scat(b)
      fire_scat(b)

      @pl.when(cur + 2 < nch)
      def _():
        fire_idx(cur + 2, b)

    fire_idx(0, 0)
    fire_idx(1, 1)

    def pair(i, _):
      step(2 * i, 0)
      step(2 * i + 1, 1)
      return 0
    lax.fori_loop(0, nch // 2, pair, 0)

    if nch % 2 == 1:
      step(nch - 1, 0)
    wait_scat(nch % 2)
    wait_scat((nch + 1) % 2)
    plsc.subcore_barrier()
    o0 = pl.multiple_of(cid * NP + sid * stripe, 128)
    o1 = pl.multiple_of(cid * NP + HP + sid * stripe, 128)
    pltpu.sync_copy(degO.at[pl.ds(z0, stripe)], out_hbm.at[pl.ds(o0, stripe)])
    pltpu.sync_copy(degI.at[pl.ds(z0, stripe)], out_hbm.at[pl.ds(o1, stripe)])

  return pl.kernel(
      body,
      out_type=jax.ShapeDtypeStruct((NC * NP,), f32),
      mesh=_mesh(),
      scratch_types=[
          pltpu.VMEM((C,), i32),
          pltpu.VMEM((C,), i32),
          pltpu.VMEM((C,), i32),
          pltpu.VMEM((C,), i32),
          pltpu.VMEM((C,), f32),
          pltpu.VMEM((stripe,), f32),
          pltpu.VMEM_SHARED((HP,), f32),
          pltpu.VMEM_SHARED((HP,), f32),
          pltpu.SemaphoreType.DMA,
          pltpu.SemaphoreType.DMA,
          pltpu.SemaphoreType.DMA,
          pltpu.SemaphoreType.DMA,
      ],
  )(src, dst)


def _edge_weight(w0, w1, gnflat, aspTX):
  E = w0.shape[0]
  DM = aspTX.shape[1]
  D = 128
  ec = E // NW
  C = 40
  nch = ec // C
  npair = nch // 2

  def body(w0_hbm, w1_hbm, gn_hbm, asp_hbm, ew_hbm,
           w0va, w0vb, w1va, w1vb, gnva, gnvb,
           ab0a, ab0b, ab1a, ab1b, ewba, ewbb,
           semi0, semi1, semg0, semg1, semo0, semo1):
    w0v = (w0va, w0vb)
    w1v = (w1va, w1vb)
    gnv = (gnva, gnvb)
    ab0 = (ab0a, ab0b)
    ab1 = (ab1a, ab1b)
    ewb = (ewba, ewbb)
    cid = lax.axis_index("c")
    sid = lax.axis_index("s")
    wid = sid * NC + cid
    iot = lax.iota(i32, L)
    kv = iot & 3
    semi = (semi0, semi1)
    semg = (semg0, semg1)
    semo = (semo0, semo1)

    def ebase(g):
      return wid * ec + g * C

    def fire_idx(g, b):
      eb = ebase(g)
      pltpu.async_copy(w0_hbm.at[pl.ds(eb, C)], w0v[b], semi[b])
      pltpu.async_copy(w1_hbm.at[pl.ds(eb, C)], w1v[b], semi[b])
      pltpu.async_copy(gn_hbm.at[pl.ds(eb * 4, C * 4)], gnv[b], semi[b])

    def wait_idx(g, b):
      eb = ebase(g)
      pltpu.make_async_copy(w0_hbm.at[pl.ds(eb, C)], w0v[b], semi[b]).wait()
      pltpu.make_async_copy(w1_hbm.at[pl.ds(eb, C)], w1v[b], semi[b]).wait()
      pltpu.make_async_copy(gn_hbm.at[pl.ds(eb * 4, C * 4)], gnv[b], semi[b]).wait()

    def fire_gather(b):
      pltpu.async_copy(asp_hbm.at[w0v[b]], ab0[b], semg[b])
      pltpu.async_copy(asp_hbm.at[w1v[b]], ab1[b], semg[b])

    def wait_gather(b):
      pltpu.make_async_copy(asp_hbm.at[w0v[b]], ab0[b], semg[b]).wait()
      pltpu.make_async_copy(asp_hbm.at[w1v[b]], ab1[b], semg[b]).wait()

    def fire_store(g, b):
      eb = pl.multiple_of(ebase(g), 8)
      pltpu.async_copy(ewb[b], ew_hbm.at[pl.ds(eb, C), :], semo[b])

    def wait_store(g, b):
      eb = pl.multiple_of(ebase(g), 8)
      pltpu.make_async_copy(ewb[b], ew_hbm.at[pl.ds(eb, C), :], semo[b]).wait()

    def compute(b):
      def edge(je, _):
        s0 = ab0[b][je, pl.ds(4 * D, L)]
        s1 = ab1[b][je, pl.ds(4 * D, L)]
        gg = gnv[b][pl.ds((je >> 2) * L, L)]
        gn_e = _perm(gg, kv + 4 * (je & 3))
        t = (s0 + s1 + gn_e) * 2.0
        m = jnp.maximum(t, _perm(t, iot ^ 1))
        m = jnp.maximum(m, _perm(m, iot ^ 2))
        p = jnp.exp(t - m)
        q = p + _perm(p, iot ^ 1)
        q = q + _perm(q, iot ^ 2)
        attn = p / q
        accs = [None] * (D // L)
        for k in range(4):
          a = attn[k]
          for d in range(D // L):
            off = k * D + d * L
            term = ab0[b][je, pl.ds(off, L)] + ab1[b][je, pl.ds(off, L)]
            accs[d] = term * a if accs[d] is None else accs[d] + term * a
        for d in range(D // L):
          ewb[b][je, pl.ds(d * L, L)] = accs[d]
        return 0
      lax.fori_loop(0, C, edge, 0)

    def step(cur, b):
      nxt = cur + 1

      @pl.when(nxt < nch)
      def _():
        wait_idx(nxt, b ^ 1)
        fire_gather(b ^ 1)
      wait_gather(b)

      @pl.when(cur >= 2)
      def _():
        wait_store(cur, b)
      compute(b)
      fire_store(cur, b)

      @pl.when(cur + 2 < nch)
      def _():
        fire_idx(cur + 2, b)

    fire_idx(0, 0)
    fire_idx(1, 1)
    wait_idx(0, 0)
    fire_gather(0)

    def pair(i, _):
      step(2 * i, 0)
      step(2 * i + 1, 1)
      return 0
    lax.fori_loop(0, npair, pair, 0)
    wait_store(nch - 2, 0)
    wait_store(nch - 1, 1)

  return pl.kernel(
      body,
      out_type=jax.ShapeDtypeStruct((E, D), f32),
      mesh=_mesh(),
      scratch_types=[
          pltpu.VMEM((C,), i32),
          pltpu.VMEM((C,), i32),
          pltpu.VMEM((C,), i32),
          pltpu.VMEM((C,), i32),
          pltpu.VMEM((4 * C,), f32),
          pltpu.VMEM((4 * C,), f32),
          pltpu.VMEM((C, DM), f32),
          pltpu.VMEM((C, DM), f32),
          pltpu.VMEM((C, DM), f32),
          pltpu.VMEM((C, DM), f32),
          pltpu.VMEM((C, D), f32),
          pltpu.VMEM((C, D), f32),
          pltpu.SemaphoreType.DMA,
          pltpu.SemaphoreType.DMA,
          pltpu.SemaphoreType.DMA,
          pltpu.SemaphoreType.DMA,
          pltpu.SemaphoreType.DMA,
          pltpu.SemaphoreType.DMA,
      ],
  )(w0, w1, gnflat, aspTX)


def _gconv_scatter(feat, ew, src, dst):
  n_nodes, D = feat.shape
  E = src.shape[0]
  ec = E // NW
  C = 40
  nch = ec // C
  NSTRIPE = 10
  rows_pt = n_nodes // NSTRIPE
  zrows = 40

  def body(feat_hbm, ew_hbm, src_hbm, dst_hbm, out_hbm,
           sidxa, sidxb, didxa, didxb, fva, fvb, eva, evb, zb, agg_sh,
           semi0, semi1, semg0, semg1, seme0, seme1, sems0, sems1):
    sidx = (sidxa, sidxb)
    didx = (didxa, didxb)
    fv = (fva, fvb)
    ev = (eva, evb)
    cid = lax.axis_index("c")
    sid = lax.axis_index("s")
    wid = sid * NC + cid

    def zfill(j, _):
      for d in range(D // L):
        zb[j, pl.ds(d * L, L)] = jnp.zeros((L,), f32)
      return 0
    lax.fori_loop(0, zrows, zfill, 0)

    @pl.when(sid < NSTRIPE)
    def _():
      for b in range(rows_pt // zrows):
        r0 = pl.multiple_of(sid * rows_pt + b * zrows, 8)
        pltpu.sync_copy(zb, agg_sh.at[pl.ds(r0, zrows), :])
    plsc.subcore_barrier()

    semi = (semi0, semi1)
    semg = (semg0, semg1)
    seme = (seme0, seme1)
    sems = (sems0, sems1)

    def ebase(g):
      return wid * ec + g * C

    def fire_idx(g, b):
      eb = ebase(g)
      pltpu.async_copy(src_hbm.at[pl.ds(eb, C)], sidx[b], semi[b])
      pltpu.async_copy(dst_hbm.at[pl.ds(eb, C)], didx[b], semi[b])

    def wait_idx(g, b):
      eb = ebase(g)
      pltpu.make_async_copy(src_hbm.at[pl.ds(eb, C)], sidx[b], semi[b]).wait()
      pltpu.make_async_copy(dst_hbm.at[pl.ds(eb, C)], didx[b], semi[b]).wait()

    def fire_gather(g, b):
      eb = pl.multiple_of(ebase(g), 8)
      pltpu.async_copy(feat_hbm.at[sidx[b]], fv[b], semg[b])
      pltpu.async_copy(ew_hbm.at[pl.ds(eb, C), :], ev[b], seme[b])

    def wait_gather(g, b):
      eb = pl.multiple_of(ebase(g), 8)
      pltpu.make_async_copy(feat_hbm.at[sidx[b]], fv[b], semg[b]).wait()
      pltpu.make_async_copy(ew_hbm.at[pl.ds(eb, C), :], ev[b], seme[b]).wait()

    def compute_scatter(b):
      def edge(jj, _):
        for u in range(2):
          je = 2 * jj + u
          for d in range(D // L):
            fv[b][je, pl.ds(d * L, L)] = (
                fv[b][je, pl.ds(d * L, L)] * ev[b][je, pl.ds(d * L, L)])
        return 0
      lax.fori_loop(0, C // 2, edge, 0)
      pltpu.async_copy(fv[b], agg_sh.at[didx[b]], sems[b], add=True)

    def wait_scat(b):
      pltpu.make_async_copy(fv[b], agg_sh.at[didx[b]], sems[b]).wait()

    def step(cur, b):
      nxt = cur + 1

      @pl.when(nxt < nch)
      def _():
        wait_idx(nxt, b ^ 1)
        fire_gather(nxt, b ^ 1)
      wait_gather(cur, b)

      @pl.when(cur >= 2)
      def _():
        wait_scat(b)
      compute_scatter(b)

      @pl.when(cur + 2 < nch)
      def _():
        fire_idx(cur + 2, b)

    fire_idx(0, 0)
    fire_idx(1, 1)
    wait_idx(0, 0)
    fire_gather(0, 0)

    def pair(i, _):
      step(2 * i, 0)
      step(2 * i + 1, 1)
      return 0
    lax.fori_loop(0, nch // 2, pair, 0)
    wait_scat(0)
    wait_scat(1)
    plsc.subcore_barrier()

    @pl.when(sid < NSTRIPE)
    def _():
      for b in range(rows_pt // zrows):
        r0 = pl.multiple_of(sid * rows_pt + b * zrows, 8)
        pltpu.sync_copy(agg_sh.at[pl.ds(r0, zrows), :],
                        out_hbm.at[cid, pl.ds(r0, zrows), :])

  return pl.kernel(
      body,
      out_type=jax.ShapeDtypeStruct((NC, n_nodes, D), f32),
      mesh=_mesh(),
      scratch_types=[
          pltpu.VMEM((C,), i32),
          pltpu.VMEM((C,), i32),
          pltpu.VMEM((C,), i32),
          pltpu.VMEM((C,), i32),
          pltpu.VMEM((C, D), f32),
          pltpu.VMEM((C, D), f32),
          pltpu.VMEM((C, D), f32),
          pltpu.VMEM((C, D), f32),
          pltpu.VMEM((zrows, D), f32),
          pltpu.VMEM_SHARED((n_nodes, D), f32),
          pltpu.SemaphoreType.DMA,
          pltpu.SemaphoreType.DMA,
          pltpu.SemaphoreType.DMA,
          pltpu.SemaphoreType.DMA,
          pltpu.SemaphoreType.DMA,
          pltpu.SemaphoreType.DMA,
          pltpu.SemaphoreType.DMA,
          pltpu.SemaphoreType.DMA,
      ],
  )(feat, ew, src, dst)


def _tc_pack(a3, attn_w2, center, odp):
  A_, N, D = a3.shape
  blk = 1000

  def body(a_ref, w_ref, c_ref, od_ref, o_ref, f_ref):
    a = a_ref[...]
    o_ref[:, 0:A_, :] = jnp.swapaxes(a, 0, 1)
    s = jnp.sum(a * w_ref[0], axis=-1)
    st = s.T
    row = jnp.concatenate(
        [st, st, st, st, jnp.zeros((blk, D - 4 * A_), f32)], axis=1)
    o_ref[:, A_:A_ + 1, :] = row[:, None, :]
    od = jnp.maximum(od_ref[:, 0:1] + od_ref[:, 1:2], 1.0)
    f_ref[...] = c_ref[...] * lax.rsqrt(od)

  return pl.pallas_call(
      body,
      grid=(N // blk,),
      in_specs=[
          pl.BlockSpec((A_, blk, D), lambda i: (0, i, 0)),
          pl.BlockSpec((1, D), lambda i: (0, 0)),
          pl.BlockSpec((blk, D), lambda i: (i, 0)),
          pl.BlockSpec((blk, 2), lambda i: (i, 0)),
      ],
      out_specs=[
          pl.BlockSpec((blk, A_ + 1, D), lambda i: (i, 0, 0)),
          pl.BlockSpec((blk, D), lambda i: (i, 0)),
      ],
      out_shape=[
          jax.ShapeDtypeStruct((N, A_ + 1, D), f32),
          jax.ShapeDtypeStruct((N, D), f32),
      ],
  )(a3, attn_w2, center, odp)


def _tc_dense(aggp, odp, idp, W, b2, g2, be2):
  _, N, D = aggp.shape

  def body(a_ref, od_ref, id_ref, w_ref, b_ref, g_ref, be_ref, o_ref):
    idn = lax.rsqrt(jnp.maximum(id_ref[:, 0:1] + id_ref[:, 1:2], 1.0))
    agg = (a_ref[0] + a_ref[1]) * idn
    y = jnp.dot(agg, w_ref[...], preferred_element_type=f32) + b_ref[...]
    mu = jnp.mean(y, axis=0, keepdims=True)
    var = jnp.mean((y - mu) ** 2, axis=0, keepdims=True)
    yn = (y - mu) * lax.rsqrt(var + 1e-5) * g_ref[...] + be_ref[...]
    yn = jnp.where(yn > 0, yn, jnp.exp(jnp.minimum(yn, 0.0)) - 1.0)
    odn = lax.rsqrt(jnp.maximum(od_ref[:, 0:1] + od_ref[:, 1:2], 1.0))
    o_ref[...] = yn * odn

  return pl.pallas_call(
      body,
      out_shape=jax.ShapeDtypeStruct((N, D), f32),
  )(aggp, odp, idp, W, b2, g2, be2)


def _tc_final(aggp, idp, W, b2, g2, be2, linW, linb2):
  _, N, D = aggp.shape
  NL = linW.shape[1]

  def body(a_ref, id_ref, w_ref, b_ref, g_ref, be_ref, lw_ref, lb_ref,
           h_ref, o2_ref):
    idn = lax.rsqrt(jnp.maximum(id_ref[:, 0:1] + id_ref[:, 1:2], 1.0))
    agg = (a_ref[0] + a_ref[1]) * idn
    y = jnp.dot(agg, w_ref[...], preferred_element_type=f32) + b_ref[...]
    mu = jnp.mean(y, axis=0, keepdims=True)
    var = jnp.mean((y - mu) ** 2, axis=0, keepdims=True)
    h = (y - mu) * lax.rsqrt(var + 1e-5) * g_ref[...] + be_ref[...]
    h_ref[...] = h
    o2_ref[...] = jnp.dot(h, lw_ref[...], preferred_element_type=f32) + lb_ref[...]

  return pl.pallas_call(
      body,
      out_shape=(
          jax.ShapeDtypeStruct((N, D), f32),
          jax.ShapeDtypeStruct((N, NL), f32),
      ),
  )(aggp, idp, W, b2, g2, be2, linW, linb2)


def kernel(edge_index, edge_w, gumbel_noise, aspect_emb, center_emb,
           attn_W, attn_b, W1, b1, W2, b2, W3, b3,
           bn_gamma, bn_beta, lin_W, lin_b):
  N, D = center_emb.shape
  A_ = gumbel_noise.shape[1]
  E = edge_index.shape[1]

  src = edge_index[0]
  dst = edge_index[1]
  w0 = edge_w[:, 0]
  w1 = edge_w[:, 1]
  a3 = aspect_emb.reshape(A_, N, D)
  gnflat = gumbel_noise.reshape(-1)

  degflat = _degrees(src, dst, N)
  NP = degflat.shape[0] // NC
  HP = NP // 2
  degout = degflat.reshape(NC, NP)
  odp = degout[:, :N].T
  idp = degout[:, HP:HP + N].T

  aspTX3, feat = _tc_pack(a3, attn_W.reshape(1, D), center_emb, odp)
  aspTX = aspTX3.reshape(N, (A_ + 1) * D)
  ew = _edge_weight(w0, w1, gnflat, aspTX)

  for (W, b) in ((W1, b1), (W2, b2)):
    aggp = _gconv_scatter(feat, ew, src, dst)
    feat = _tc_dense(aggp, odp, idp, W, b.reshape(1, D),
                     bn_gamma.reshape(1, D), bn_beta.reshape(1, D))
  aggp = _gconv_scatter(feat, ew, src, dst)
  h, out2 = _tc_final(aggp, idp, W3, b3.reshape(1, D),
                      bn_gamma.reshape(1, D), bn_beta.reshape(1, D),
                      lin_W, lin_b.reshape(1, lin_W.shape[1]))
  return (h, out2)
